# Initial kernel scaffold; baseline (speedup 1.0000x reference)
import functools

import jax
import jax.numpy as jnp
from jax.experimental import pallas as pl
from jax.experimental.pallas import tpu as pltpu
from jax.experimental.pallas import tpu_sc as plsc

_NUM_BINS = 1000000
_HASH_MULT = 2654435761
_EMBED_DIM = 32
_WINDOW = 128


def _hash_body(x_ref, o_ref):
    h = x_ref[...].astype(jnp.uint32) * jnp.uint32(_HASH_MULT)
    o_ref[...] = (h % jnp.uint32(_NUM_BINS)).astype(jnp.int32)


def _sc_gather(table, idx_flat, n_idx):
    mesh = plsc.VectorSubcoreMesh(core_axis_name="core", subcore_axis_name="subcore")
    out_type = jax.ShapeDtypeStruct((n_idx, _EMBED_DIM), table.dtype)

    @functools.partial(pl.kernel, out_type=out_type, mesh=mesh)
    def k(table_hbm, idx_hbm, out_hbm):
        def body(i_vmem, o_vmem):
            pltpu.sync_copy(table_hbm.at[i_vmem.at[0]], o_vmem)

        pltpu.emit_pipeline(
            body,
            grid=(n_idx // _WINDOW,),
            in_specs=[pl.BlockSpec((1, _WINDOW), lambda i: (0, i))],
            out_specs=[pl.BlockSpec((_WINDOW, _EMBED_DIM), lambda i: (i, 0))],
            core_axis_name=("core", "subcore"),
            dimension_semantics=(pltpu.PARALLEL,),
        )(idx_hbm, out_hbm)

    return k(table, idx_flat)


def kernel(inputs, table):
    b, f = inputs.shape
    n = b * f
    flat = inputs.reshape(n // 128, 128)
    idx = pl.pallas_call(
        _hash_body,
        out_shape=jax.ShapeDtypeStruct(flat.shape, jnp.int32),
    )(flat)
    out = _sc_gather(table, idx.reshape(1, n), n)
    return out.reshape(b, f, _EMBED_DIM)


# speedup vs baseline: 1.4672x; 1.4672x over previous
import functools

import jax
import jax.numpy as jnp
from jax.experimental import pallas as pl
from jax.experimental.pallas import tpu as pltpu
from jax.experimental.pallas import tpu_sc as plsc

_NUM_BINS = 1000000
_HASH_MULT = 2654435761
_EMBED_DIM = 32
_WINDOW = 128


def _hash_body(x_ref, o_ref):
    h = x_ref[...].astype(jnp.uint32) * jnp.uint32(_HASH_MULT)
    o_ref[...] = (h % jnp.uint32(_NUM_BINS)).astype(jnp.int32)


def _sc_gather(table, idx_flat, n_idx):
    mesh = plsc.VectorSubcoreMesh(core_axis_name="core", subcore_axis_name="subcore")
    out_type = jax.ShapeDtypeStruct((n_idx, _EMBED_DIM), table.dtype)

    @functools.partial(
        pl.kernel,
        out_type=out_type,
        mesh=mesh,
        compiler_params=pltpu.CompilerParams(use_tc_tiling_on_sc=False),
    )
    def k(table_hbm, idx_hbm, out_hbm):
        def body(i_vmem, o_vmem):
            pltpu.sync_copy(table_hbm.at[i_vmem.at[0]], o_vmem)

        pltpu.emit_pipeline(
            body,
            grid=(n_idx // _WINDOW,),
            in_specs=[pl.BlockSpec((1, _WINDOW), lambda i: (0, i))],
            out_specs=[pl.BlockSpec((_WINDOW, _EMBED_DIM), lambda i: (i, 0))],
            core_axis_name=("core", "subcore"),
            dimension_semantics=(pltpu.PARALLEL,),
        )(idx_hbm, out_hbm)

    return k(table, idx_flat)


def kernel(inputs, table):
    b, f = inputs.shape
    n = b * f
    flat = inputs.reshape(n // 128, 128)
    idx = pl.pallas_call(
        _hash_body,
        out_shape=jax.ShapeDtypeStruct(flat.shape, jnp.int32),
    )(flat)
    out = _sc_gather(table, idx.reshape(1, n), n)
    return out.reshape(b, f, _EMBED_DIM)


# speedup vs baseline: 1.5673x; 1.0682x over previous
import functools

import jax
import jax.numpy as jnp
from jax.experimental import pallas as pl
from jax.experimental.pallas import tpu as pltpu
from jax.experimental.pallas import tpu_sc as plsc

_NUM_BINS = 1000000
_HASH_MULT = 2654435761
_EMBED_DIM = 32
_WINDOW = 128
_K = 13


def _hash_body(x_ref, o_ref):
    h = x_ref[...].astype(jnp.uint32) * jnp.uint32(_HASH_MULT)
    o_ref[...] = (h % jnp.uint32(_NUM_BINS)).astype(jnp.int32)


def _sc_gather(table, idx2d, n_idx):
    mesh = plsc.VectorSubcoreMesh(core_axis_name="core", subcore_axis_name="subcore")
    out_type = jax.ShapeDtypeStruct((n_idx, _EMBED_DIM), table.dtype)

    @functools.partial(
        pl.kernel,
        out_type=out_type,
        mesh=mesh,
        scratch_types=[pltpu.SemaphoreType.DMA],
        compiler_params=pltpu.CompilerParams(use_tc_tiling_on_sc=False),
    )
    def k(table_hbm, idx_hbm, out_hbm, sem):
        def body(i_vmem, o_vmem):
            copies = [
                pltpu.async_copy(
                    table_hbm.at[i_vmem.at[j]],
                    o_vmem.at[pl.ds(j * _WINDOW, _WINDOW)],
                    sem,
                )
                for j in range(_K)
            ]
            for c in copies:
                c.wait()

        pltpu.emit_pipeline(
            body,
            grid=(n_idx // (_K * _WINDOW),),
            in_specs=[pl.BlockSpec((_K, _WINDOW), lambda i: (i, 0))],
            out_specs=[pl.BlockSpec((_K * _WINDOW, _EMBED_DIM), lambda i: (i, 0))],
            core_axis_name=("core", "subcore"),
            dimension_semantics=(pltpu.PARALLEL,),
        )(idx_hbm, out_hbm)

    return k(table, idx2d)


def kernel(inputs, table):
    b, f = inputs.shape
    n = b * f
    flat = inputs.reshape(n // _WINDOW, _WINDOW)
    idx = pl.pallas_call(
        _hash_body,
        out_shape=jax.ShapeDtypeStruct(flat.shape, jnp.int32),
    )(flat)
    out = _sc_gather(table, idx, n)
    return out.reshape(b, f, _EMBED_DIM)
